# baseline (speedup 1.0000x reference)
import functools

import jax
import jax.numpy as jnp
from jax import lax
from jax.experimental import pallas as pl
from jax.experimental.pallas import tpu as pltpu
from jax.experimental.pallas import tpu_sc as plsc

B = 16384
K = 16
ROWS_PER_SUPER = 8

_INFO = plsc.get_sparse_core_info()
_NC, _NS, _L = _INFO.num_cores, _INFO.num_subcores, _INFO.num_lanes
_NW = _NC * _NS
_BPW = B // _NW
_CH = 128
_NCH = _BPW // _CH
_GPC = _CH // _L

_BITREV = [((j & 1) << 3) | ((j & 2) << 1) | ((j & 4) >> 1) | ((j & 8) >> 3)
           for j in range(16)]


def _sc_body(users_hbm, items_hbm, gu_hbm, gi_hbm,
             xui_out, gu_out, gi_out,
             uidx_v, iidx_v, usidx_v, isidx_v,
             usupa_v, isupa_v, usupb_v, isupb_v,
             urows_v, irows_v, xui_v,
             gsema, gsemb, osem):
    gu_sup = gu_hbm
    gi_sup = gi_hbm
    wid = lax.axis_index("s") * _NC + lax.axis_index("c")
    base = wid * _BPW

    pltpu.sync_copy(users_hbm.at[pl.ds(base, _BPW)], uidx_v)
    pltpu.sync_copy(items_hbm.at[pl.ds(base, _BPW)], iidx_v)

    for t in range(_BPW // _L):
        sl = pl.ds(t * _L, _L)
        usidx_v[sl] = uidx_v[sl] >> 3
        isidx_v[sl] = iidx_v[sl] >> 3

    bufs = [(usupa_v, isupa_v, gsema), (usupb_v, isupb_v, gsemb)]

    def fire(c):
        usup, isup, sem = bufs[c % 2]
        sl = pl.ds(c * _CH, _CH)
        return [
            pltpu.async_copy(gu_sup.at[usidx_v.at[sl]], usup, sem),
            pltpu.async_copy(gi_sup.at[isidx_v.at[sl]], isup, sem),
        ]

    lanes = lax.iota(jnp.int32, _L)
    stages = [((lanes & d) == 0, lanes ^ d) for d in (8, 4, 2, 1)]

    pending = {0: fire(0)}
    outs = []
    for c in range(_NCH):
        if c + 1 < _NCH:
            pending[c + 1] = fire(c + 1)
        for cp in pending.pop(c):
            cp.wait()
        usup, isup, _ = bufs[c % 2]

        def group(g, carry, c=c, usup=usup, isup=isup):
            r0 = c * _CH + g * _L
            uoffs = (uidx_v[pl.ds(r0, _L)] & 7) * K
            ioffs = (iidx_v[pl.ds(r0, _L)] & 7) * K
            vecs = []
            for j in range(_L):
                jb = g * _L + _BITREV[j]
                r = c * _CH + jb
                uoff = uoffs[_BITREV[j]]
                ioff = ioffs[_BITREV[j]]
                u16 = usup[jb, pl.ds(uoff, K)]
                i16 = isup[jb, pl.ds(ioff, K)]
                urows_v[r, :] = u16
                irows_v[r, :] = i16
                vecs.append(u16 * i16)
            for m, sw in stages:
                nxt = []
                for a, b in zip(vecs[0::2], vecs[1::2]):
                    nxt.append(jnp.where(m, a, b) +
                               jnp.where(m, b, a).at[sw].get(
                                   mode="promise_in_bounds",
                                   unique_indices=True))
                vecs = nxt
            xui_v[pl.ds(c * _CH + g * _L, _L)] = vecs[0]
            return carry

        lax.fori_loop(0, _GPC, group, 0)

        csl = pl.ds(c * _CH, _CH)
        outs.append(pltpu.async_copy(
            urows_v.at[csl, :], gu_out.at[pl.ds(base + c * _CH, _CH), :], osem))
        outs.append(pltpu.async_copy(
            irows_v.at[csl, :], gi_out.at[pl.ds(base + c * _CH, _CH), :], osem))

    pltpu.sync_copy(xui_v, xui_out.at[pl.ds(base, _BPW)])
    for cp in outs:
        cp.wait()


_mf_kernel = functools.partial(
    pl.kernel,
    mesh=plsc.VectorSubcoreMesh(core_axis_name="c", subcore_axis_name="s"),
    out_type=(
        jax.ShapeDtypeStruct((B,), jnp.float32),
        jax.ShapeDtypeStruct((B, K), jnp.float32),
        jax.ShapeDtypeStruct((B, K), jnp.float32),
    ),
    scratch_types=[
        pltpu.VMEM((_BPW,), jnp.int32),
        pltpu.VMEM((_BPW,), jnp.int32),
        pltpu.VMEM((_BPW,), jnp.int32),
        pltpu.VMEM((_BPW,), jnp.int32),
        pltpu.VMEM((_CH, ROWS_PER_SUPER * K), jnp.float32),
        pltpu.VMEM((_CH, ROWS_PER_SUPER * K), jnp.float32),
        pltpu.VMEM((_CH, ROWS_PER_SUPER * K), jnp.float32),
        pltpu.VMEM((_CH, ROWS_PER_SUPER * K), jnp.float32),
        pltpu.VMEM((_BPW, K), jnp.float32),
        pltpu.VMEM((_BPW, K), jnp.float32),
        pltpu.VMEM((_BPW,), jnp.float32),
        pltpu.SemaphoreType.DMA,
        pltpu.SemaphoreType.DMA,
        pltpu.SemaphoreType.DMA,
    ],
    compiler_params=pltpu.CompilerParams(use_tc_tiling_on_sc=False),
)(_sc_body)


def kernel(users, items, Gu, Gi):
    gu_sup = Gu.reshape(-1, ROWS_PER_SUPER * K)
    gi_sup = Gi.reshape(-1, ROWS_PER_SUPER * K)
    xui, gamma_u, gamma_i = _mf_kernel(
        users.astype(jnp.int32), items.astype(jnp.int32), gu_sup, gi_sup)
    return (xui, gamma_u, gamma_i)


# speedup vs baseline: 6.2564x; 6.2564x over previous
import functools

import jax
import jax.numpy as jnp
from jax import lax
from jax.experimental import pallas as pl
from jax.experimental.pallas import tpu as pltpu
from jax.experimental.pallas import tpu_sc as plsc

B = 16384
K = 16
TW = 128

_INFO = plsc.get_sparse_core_info()
_NC, _NS, _L = _INFO.num_cores, _INFO.num_subcores, _INFO.num_lanes
_NW = _NC * _NS
_BPW = B // _NW
_NG = _BPW // _L


def _sc_body(users_hbm, items_hbm, gut_hbm, git_hbm,
             xui_out, guo, gio,
             uidx_v, iidx_v, ubuf_v, ibuf_v,
             ustage_v, istage_v, xui_v, gsem, osem):
    wid = lax.axis_index("s") * _NC + lax.axis_index("c")
    base = wid * _BPW

    pltpu.sync_copy(users_hbm.at[pl.ds(base, _BPW)], uidx_v)
    pltpu.sync_copy(items_hbm.at[pl.ds(base, _BPW)], iidx_v)

    lanes = lax.iota(jnp.int32, _L)

    def group(t, carry):
        sl = pl.ds(t * _L, _L)
        uids = uidx_v[sl]
        iids = iidx_v[sl]
        cps = []
        for slot in range(_L):
            utc = pl.multiple_of((uids[slot] >> 7) * TW, TW)
            itc = pl.multiple_of((iids[slot] >> 7) * TW, TW)
            cps.append(pltpu.async_copy(
                gut_hbm.at[:, pl.ds(utc, TW)], ubuf_v.at[slot], gsem))
            cps.append(pltpu.async_copy(
                git_hbm.at[:, pl.ds(itc, TW)], ibuf_v.at[slot], gsem))
        for cp in cps:
            cp.wait()

        ul = uids & (TW - 1)
        il = iids & (TW - 1)
        acc = None
        for k in range(K):
            kf = jnp.full((_L,), k, jnp.int32)
            uvals = plsc.load_gather(ubuf_v, [lanes, kf, ul])
            ivals = plsc.load_gather(ibuf_v, [lanes, kf, il])
            ustage_v[k, sl] = uvals
            istage_v[k, sl] = ivals
            p = uvals * ivals
            acc = p if acc is None else acc + p
        xui_v[sl] = acc
        return carry

    lax.fori_loop(0, _NG, group, 0)

    obase = pl.multiple_of(base, 128)
    outs = [
        pltpu.async_copy(ustage_v, guo.at[:, pl.ds(obase, _BPW)], osem),
        pltpu.async_copy(istage_v, gio.at[:, pl.ds(obase, _BPW)], osem),
    ]
    pltpu.sync_copy(xui_v, xui_out.at[pl.ds(base, _BPW)])
    for cp in outs:
        cp.wait()


_mf_kernel = functools.partial(
    pl.kernel,
    mesh=plsc.VectorSubcoreMesh(core_axis_name="c", subcore_axis_name="s"),
    out_type=(
        jax.ShapeDtypeStruct((B,), jnp.float32),
        jax.ShapeDtypeStruct((K, B), jnp.float32),
        jax.ShapeDtypeStruct((K, B), jnp.float32),
    ),
    scratch_types=[
        pltpu.VMEM((_BPW,), jnp.int32),
        pltpu.VMEM((_BPW,), jnp.int32),
        pltpu.VMEM((_L, K, TW), jnp.float32),
        pltpu.VMEM((_L, K, TW), jnp.float32),
        pltpu.VMEM((K, _BPW), jnp.float32),
        pltpu.VMEM((K, _BPW), jnp.float32),
        pltpu.VMEM((_BPW,), jnp.float32),
        pltpu.SemaphoreType.DMA,
        pltpu.SemaphoreType.DMA,
    ],
    compiler_params=pltpu.CompilerParams(needs_layout_passes=False),
)(_sc_body)


def kernel(users, items, Gu, Gi):
    xui, guo, gio = _mf_kernel(
        users.astype(jnp.int32), items.astype(jnp.int32), Gu.T, Gi.T)
    return (xui, guo.T, gio.T)
